# baseline (speedup 1.0000x reference)
import functools

import jax
import jax.numpy as jnp
from jax import lax
from jax.experimental import pallas as pl

NUM_PTS = 2048
HF = 128
WF = 128
HC = 32
WC = 32
NC = 19
CF = 192
NPIX = HF * WF
BLK = 8192
NBLK = NPIX // BLK


def _upsample_matrices():
    def mat(out_n, in_n):
        s = jnp.linspace(0.0, in_n - 1.0, out_n)
        i0 = jnp.clip(jnp.floor(s), 0, in_n - 1)
        i1 = jnp.clip(i0 + 1, 0, in_n - 1)
        w = (s - i0)[:, None]
        oh0 = jax.nn.one_hot(i0.astype(jnp.int32), in_n, dtype=jnp.float32)
        oh1 = jax.nn.one_hot(i1.astype(jnp.int32), in_n, dtype=jnp.float32)
        return oh0 * (1.0 - w) + oh1 * w

    wy = mat(HF, HC)
    wxt = mat(WF, WC).T
    return wy, wxt


def _select_kernel(coarse_ref, wy_ref, wxt_ref, out_ref):
    wy = wy_ref[...]
    wxt = wxt_ref[...]
    ups = []
    for ci in range(NC):
        a1 = jnp.dot(coarse_ref[0, ci], wxt, precision=lax.Precision.HIGHEST,
                     preferred_element_type=jnp.float32)
        m = jnp.dot(wy, a1, precision=lax.Precision.HIGHEST,
                    preferred_element_type=jnp.float32)
        ups.append(m)
        out_ref[0, ci] = m
    cmax = functools.reduce(jnp.maximum, ups)
    s = functools.reduce(jnp.add, [jnp.exp(u - cmax) for u in ups])
    unc = -(1.0 / s)
    keys = -lax.bitcast_convert_type(unc, jnp.int32)

    kpts = jnp.int32(NUM_PTS)

    def count_ge(t):
        return jnp.sum((keys >= t).astype(jnp.int32))

    def body_val(_, carry):
        lo, hi = carry
        mid = lo + (hi - lo) // 2
        ge = count_ge(mid) >= kpts
        return (jnp.where(ge, mid, lo), jnp.where(ge, hi, mid))

    lo0 = jnp.int32(1082130432)
    hi0 = jnp.int32(2147483647)
    lo, hi = lax.fori_loop(0, 31, body_val, (lo0, hi0))
    kth = lo
    n_gt = count_ge(kth + 1)
    need = kpts - n_gt

    iy = lax.broadcasted_iota(jnp.int32, (HF, WF), 0)
    ix = lax.broadcasted_iota(jnp.int32, (HF, WF), 1)
    idx = iy * WF + ix
    eq = keys == kth

    def count_eq_le(j):
        return jnp.sum((eq & (idx <= j)).astype(jnp.int32))

    def body_idx(_, carry):
        lo2, hi2 = carry
        mid = lo2 + (hi2 - lo2) // 2
        ge = count_eq_le(mid) >= need
        return (jnp.where(ge, lo2, mid), jnp.where(ge, mid, hi2))

    lo2, hi2 = lax.fori_loop(0, 15, body_idx, (jnp.int32(-1), jnp.int32(NPIX - 1)))
    jthr = hi2
    mask = (keys > kth) | (eq & (idx <= jthr))
    out_ref[0, NC] = mask.astype(jnp.float32)


def _mlp_kernel(fine_ref, aux_ref, w1_ref, w2_ref, w3_ref, b1_ref, b2_ref,
                b3_ref, out_ref):
    fine = fine_ref[0]
    aux = aux_ref[0]
    cu = aux[:NC]
    msk = aux[NC:NC + 1]
    pad = jnp.zeros((256 - CF - NC, BLK), jnp.bfloat16)
    x = jnp.concatenate([fine.astype(jnp.bfloat16), cu.astype(jnp.bfloat16),
                         pad], axis=0)
    h = jnp.dot(w1_ref[...].astype(jnp.bfloat16), x,
                preferred_element_type=jnp.float32)
    h = jnp.maximum(h + b1_ref[...], 0.0)
    h = jnp.dot(w2_ref[...].astype(jnp.bfloat16), h.astype(jnp.bfloat16),
                preferred_element_type=jnp.float32)
    h = jnp.maximum(h + b2_ref[...], 0.0)
    y = jnp.dot(w3_ref[...].astype(jnp.bfloat16), h.astype(jnp.bfloat16),
                preferred_element_type=jnp.float32)
    y = y + b3_ref[...]
    out_ref[0] = jnp.where(msk > 0.0, y, cu)


def kernel(coarse_logits, fine_features, W1, b1, W2, b2, W3, b3):
    B = coarse_logits.shape[0]
    wy, wxt = _upsample_matrices()

    aux4 = pl.pallas_call(
        _select_kernel,
        grid=(B,),
        in_specs=[
            pl.BlockSpec((1, NC, HC, WC), lambda b: (b, 0, 0, 0)),
            pl.BlockSpec((HF, HC), lambda b: (0, 0)),
            pl.BlockSpec((HC, WF), lambda b: (0, 0)),
        ],
        out_specs=pl.BlockSpec((1, NC + 1, HF, WF), lambda b: (b, 0, 0, 0)),
        out_shape=jax.ShapeDtypeStruct((B, NC + 1, HF, WF), jnp.float32),
    )(coarse_logits, wy, wxt)

    aux = aux4.reshape(B, NC + 1, NPIX)
    fine2 = fine_features.reshape(B, CF, NPIX)
    w1p = jnp.pad(W1, ((0, 0), (0, 256 - W1.shape[1])))
    b1r = b1[:, None]
    b2r = b2[:, None]
    b3r = b3[:, None]

    out = pl.pallas_call(
        _mlp_kernel,
        grid=(B, NBLK),
        in_specs=[
            pl.BlockSpec((1, CF, BLK), lambda b, j: (b, 0, j)),
            pl.BlockSpec((1, NC + 1, BLK), lambda b, j: (b, 0, j)),
            pl.BlockSpec((256, 256), lambda b, j: (0, 0)),
            pl.BlockSpec((256, 256), lambda b, j: (0, 0)),
            pl.BlockSpec((NC, 256), lambda b, j: (0, 0)),
            pl.BlockSpec((256, 1), lambda b, j: (0, 0)),
            pl.BlockSpec((256, 1), lambda b, j: (0, 0)),
            pl.BlockSpec((NC, 1), lambda b, j: (0, 0)),
        ],
        out_specs=pl.BlockSpec((1, NC, BLK), lambda b, j: (b, 0, j)),
        out_shape=jax.ShapeDtypeStruct((B, NC, NPIX), jnp.float32),
    )(fine2, aux, w1p, W2, W3, b1r, b2r, b3r)

    return out.reshape(B, NC, HF, WF)


# speedup vs baseline: 1.5069x; 1.5069x over previous
import functools

import jax
import jax.numpy as jnp
from jax import lax
from jax.experimental import pallas as pl

NUM_PTS = 2048
HF = 128
WF = 128
HC = 32
WC = 32
NC = 19
CF = 192
NPIX = HF * WF
BLK = 8192
NBLK = NPIX // BLK


def _upsample_matrices():
    def mat(out_n, in_n):
        s = jnp.linspace(0.0, in_n - 1.0, out_n)
        i0 = jnp.clip(jnp.floor(s), 0, in_n - 1)
        i1 = jnp.clip(i0 + 1, 0, in_n - 1)
        w = (s - i0)[:, None]
        oh0 = jax.nn.one_hot(i0.astype(jnp.int32), in_n, dtype=jnp.float32)
        oh1 = jax.nn.one_hot(i1.astype(jnp.int32), in_n, dtype=jnp.float32)
        return oh0 * (1.0 - w) + oh1 * w

    wy = mat(HF, HC)
    wxt = mat(WF, WC).T
    return wy, wxt


def _select_kernel(coarse_ref, wy_ref, wxt_ref, out_ref):
    wy = wy_ref[...]
    wxt = wxt_ref[...]
    ups = []
    for ci in range(NC):
        a1 = jnp.dot(coarse_ref[0, ci], wxt, precision=lax.Precision.HIGHEST,
                     preferred_element_type=jnp.float32)
        m = jnp.dot(wy, a1, precision=lax.Precision.HIGHEST,
                    preferred_element_type=jnp.float32)
        ups.append(m)
        out_ref[0, ci] = m
    cmax = functools.reduce(jnp.maximum, ups)
    s = functools.reduce(jnp.add, [jnp.exp(u - cmax) for u in ups])
    unc = -(1.0 / s)
    keys = -lax.bitcast_convert_type(unc, jnp.int32)

    kpts = jnp.int32(NUM_PTS)

    def count_ge(t):
        return jnp.sum((keys >= t).astype(jnp.int32))

    def body_val(_, carry):
        lo, hi = carry
        mid = lo + (hi - lo) // 2
        ge = count_ge(mid) >= kpts
        return (jnp.where(ge, mid, lo), jnp.where(ge, hi, mid))

    lo0 = jnp.int32(1082130432)
    hi0 = jnp.int32(2147483647)
    lo, hi = lax.fori_loop(0, 31, body_val, (lo0, hi0))
    kth = lo
    n_gt = count_ge(kth + 1)
    need = kpts - n_gt

    iy = lax.broadcasted_iota(jnp.int32, (HF, WF), 0)
    ix = lax.broadcasted_iota(jnp.int32, (HF, WF), 1)
    idx = iy * WF + ix
    eq = keys == kth

    def count_eq_le(j):
        return jnp.sum((eq & (idx <= j)).astype(jnp.int32))

    def body_idx(_, carry):
        lo2, hi2 = carry
        mid = lo2 + (hi2 - lo2) // 2
        ge = count_eq_le(mid) >= need
        return (jnp.where(ge, lo2, mid), jnp.where(ge, mid, hi2))

    lo2, hi2 = lax.fori_loop(0, 15, body_idx, (jnp.int32(-1), jnp.int32(NPIX - 1)))
    jthr = hi2
    mask = (keys > kth) | (eq & (idx <= jthr))
    out_ref[0, NC] = mask.astype(jnp.float32)


def _mlp_kernel(fine_ref, aux_ref, w1_ref, w2_ref, w3_ref, b1_ref, b2_ref,
                b3_ref, out_ref):
    fine = fine_ref[0].reshape(CF, BLK)
    aux = aux_ref[0].reshape(NC + 1, BLK)
    cu = aux[:NC]
    msk = aux[NC:NC + 1]
    pad = jnp.zeros((256 - CF - NC, BLK), jnp.bfloat16)
    x = jnp.concatenate([fine.astype(jnp.bfloat16), cu.astype(jnp.bfloat16),
                         pad], axis=0)
    h = jnp.dot(w1_ref[...].astype(jnp.bfloat16), x,
                preferred_element_type=jnp.float32)
    h = jnp.maximum(h + b1_ref[...], 0.0)
    h = jnp.dot(w2_ref[...].astype(jnp.bfloat16), h.astype(jnp.bfloat16),
                preferred_element_type=jnp.float32)
    h = jnp.maximum(h + b2_ref[...], 0.0)
    y = jnp.dot(w3_ref[...].astype(jnp.bfloat16), h.astype(jnp.bfloat16),
                preferred_element_type=jnp.float32)
    y = y + b3_ref[...]
    out_ref[0] = jnp.where(msk > 0.0, y, cu).reshape(NC, BLK // WF, WF)


def kernel(coarse_logits, fine_features, W1, b1, W2, b2, W3, b3):
    B = coarse_logits.shape[0]
    wy, wxt = _upsample_matrices()

    aux4 = pl.pallas_call(
        _select_kernel,
        grid=(B,),
        in_specs=[
            pl.BlockSpec((1, NC, HC, WC), lambda b: (b, 0, 0, 0)),
            pl.BlockSpec((HF, HC), lambda b: (0, 0)),
            pl.BlockSpec((HC, WF), lambda b: (0, 0)),
        ],
        out_specs=pl.BlockSpec((1, NC + 1, HF, WF), lambda b: (b, 0, 0, 0)),
        out_shape=jax.ShapeDtypeStruct((B, NC + 1, HF, WF), jnp.float32),
    )(coarse_logits, wy, wxt)

    w1p = jnp.pad(W1, ((0, 0), (0, 256 - W1.shape[1])))
    b1r = b1[:, None]
    b2r = b2[:, None]
    b3r = b3[:, None]

    out = pl.pallas_call(
        _mlp_kernel,
        grid=(B, NBLK),
        in_specs=[
            pl.BlockSpec((1, CF, BLK // WF, WF), lambda b, j: (b, 0, j, 0)),
            pl.BlockSpec((1, NC + 1, BLK // WF, WF), lambda b, j: (b, 0, j, 0)),
            pl.BlockSpec((256, 256), lambda b, j: (0, 0)),
            pl.BlockSpec((256, 256), lambda b, j: (0, 0)),
            pl.BlockSpec((NC, 256), lambda b, j: (0, 0)),
            pl.BlockSpec((256, 1), lambda b, j: (0, 0)),
            pl.BlockSpec((256, 1), lambda b, j: (0, 0)),
            pl.BlockSpec((NC, 1), lambda b, j: (0, 0)),
        ],
        out_specs=pl.BlockSpec((1, NC, BLK // WF, WF), lambda b, j: (b, 0, j, 0)),
        out_shape=jax.ShapeDtypeStruct((B, NC, HF, WF), jnp.float32),
    )(fine_features, aux4, w1p, W2, W3, b1r, b2r, b3r)

    return out


# speedup vs baseline: 1.6363x; 1.0859x over previous
import functools

import jax
import jax.numpy as jnp
from jax import lax
from jax.experimental import pallas as pl

NUM_PTS = 2048
HF = 128
WF = 128
HC = 32
WC = 32
NC = 19
CF = 192
NPIX = HF * WF
BLK = 8192
NBLK = NPIX // BLK


def _upsample_matrices():
    def mat(out_n, in_n):
        s = jnp.linspace(0.0, in_n - 1.0, out_n)
        i0 = jnp.clip(jnp.floor(s), 0, in_n - 1)
        i1 = jnp.clip(i0 + 1, 0, in_n - 1)
        w = (s - i0)[:, None]
        oh0 = jax.nn.one_hot(i0.astype(jnp.int32), in_n, dtype=jnp.float32)
        oh1 = jax.nn.one_hot(i1.astype(jnp.int32), in_n, dtype=jnp.float32)
        return oh0 * (1.0 - w) + oh1 * w

    wy = mat(HF, HC)
    wxt = mat(WF, WC).T
    return wy, wxt


def _select_kernel(coarse_ref, wy_ref, wxt_ref, out_ref):
    wy = wy_ref[...]
    wxt = wxt_ref[...]
    ups = []
    for ci in range(NC):
        a1 = jnp.dot(coarse_ref[0, ci], wxt, precision=lax.Precision.HIGHEST,
                     preferred_element_type=jnp.float32)
        m = jnp.dot(wy, a1, precision=lax.Precision.HIGHEST,
                    preferred_element_type=jnp.float32)
        ups.append(m)
        out_ref[0, ci] = m
    cmax = functools.reduce(jnp.maximum, ups)
    s = functools.reduce(jnp.add, [jnp.exp(u - cmax) for u in ups])
    unc = -(1.0 / s)
    keys = -lax.bitcast_convert_type(unc, jnp.int32)

    kpts = jnp.int32(NUM_PTS)
    kflat = keys.reshape(1, NPIX)
    io16 = lax.broadcasted_iota(jnp.int32, (16, 1), 0) + 1

    lo = jnp.min(keys)
    hi = jnp.max(keys) + 1

    def body_val(_, carry):
        lo, hi = carry
        step = (hi - lo + 15) // 16
        ts = lo + step * io16
        cnt = jnp.sum((kflat >= ts).astype(jnp.int32), axis=1)
        num_ok = jnp.sum((cnt >= kpts).astype(jnp.int32))
        return (lo + num_ok * step,
                jnp.minimum(hi, lo + (num_ok + 1) * step))

    lo, hi = lax.fori_loop(0, 7, body_val, (lo, hi))
    kth = lo
    n_gt = jnp.sum((keys >= kth + 1).astype(jnp.int32))
    need = kpts - n_gt

    iy = lax.broadcasted_iota(jnp.int32, (HF, WF), 0)
    ix = lax.broadcasted_iota(jnp.int32, (HF, WF), 1)
    idx = iy * WF + ix
    eq = keys == kth
    eqidx = jnp.where(eq, idx, jnp.int32(NPIX)).reshape(1, NPIX)

    def body_idx(_, carry):
        lo2, hi2 = carry
        step = (hi2 - lo2 + 15) // 16
        ts = lo2 + step * io16
        cnt = jnp.sum((eqidx <= ts).astype(jnp.int32), axis=1)
        num_lt = jnp.sum((cnt < need).astype(jnp.int32))
        return (lo2 + num_lt * step,
                jnp.minimum(hi2, lo2 + (num_lt + 1) * step))

    lo2, hi2 = lax.fori_loop(0, 4, body_idx,
                             (jnp.int32(-1), jnp.int32(NPIX - 1)))
    jthr = hi2
    mask = (keys > kth) | (eq & (idx <= jthr))
    out_ref[0, NC] = mask.astype(jnp.float32)


def _mlp_kernel(fine_ref, aux_ref, w1_ref, w2_ref, w3_ref, b1_ref, b2_ref,
                b3_ref, out_ref):
    fine = fine_ref[0].reshape(CF, BLK)
    aux = aux_ref[0].reshape(NC + 1, BLK)
    cu = aux[:NC]
    msk = aux[NC:NC + 1]
    pad = jnp.zeros((256 - CF - NC, BLK), jnp.bfloat16)
    x = jnp.concatenate([fine.astype(jnp.bfloat16), cu.astype(jnp.bfloat16),
                         pad], axis=0)
    h = jnp.dot(w1_ref[...].astype(jnp.bfloat16), x,
                preferred_element_type=jnp.float32)
    h = jnp.maximum(h + b1_ref[...], 0.0)
    h = jnp.dot(w2_ref[...].astype(jnp.bfloat16), h.astype(jnp.bfloat16),
                preferred_element_type=jnp.float32)
    h = jnp.maximum(h + b2_ref[...], 0.0)
    y = jnp.dot(w3_ref[...].astype(jnp.bfloat16), h.astype(jnp.bfloat16),
                preferred_element_type=jnp.float32)
    y = y + b3_ref[...]
    out_ref[0] = jnp.where(msk > 0.0, y, cu).reshape(NC, BLK // WF, WF)


def kernel(coarse_logits, fine_features, W1, b1, W2, b2, W3, b3):
    B = coarse_logits.shape[0]
    wy, wxt = _upsample_matrices()

    aux4 = pl.pallas_call(
        _select_kernel,
        grid=(B,),
        in_specs=[
            pl.BlockSpec((1, NC, HC, WC), lambda b: (b, 0, 0, 0)),
            pl.BlockSpec((HF, HC), lambda b: (0, 0)),
            pl.BlockSpec((HC, WF), lambda b: (0, 0)),
        ],
        out_specs=pl.BlockSpec((1, NC + 1, HF, WF), lambda b: (b, 0, 0, 0)),
        out_shape=jax.ShapeDtypeStruct((B, NC + 1, HF, WF), jnp.float32),
    )(coarse_logits, wy, wxt)

    w1p = jnp.pad(W1, ((0, 0), (0, 256 - W1.shape[1])))
    b1r = b1[:, None]
    b2r = b2[:, None]
    b3r = b3[:, None]

    out = pl.pallas_call(
        _mlp_kernel,
        grid=(B, NBLK),
        in_specs=[
            pl.BlockSpec((1, CF, BLK // WF, WF), lambda b, j: (b, 0, j, 0)),
            pl.BlockSpec((1, NC + 1, BLK // WF, WF), lambda b, j: (b, 0, j, 0)),
            pl.BlockSpec((256, 256), lambda b, j: (0, 0)),
            pl.BlockSpec((256, 256), lambda b, j: (0, 0)),
            pl.BlockSpec((NC, 256), lambda b, j: (0, 0)),
            pl.BlockSpec((256, 1), lambda b, j: (0, 0)),
            pl.BlockSpec((256, 1), lambda b, j: (0, 0)),
            pl.BlockSpec((NC, 1), lambda b, j: (0, 0)),
        ],
        out_specs=pl.BlockSpec((1, NC, BLK // WF, WF), lambda b, j: (b, 0, j, 0)),
        out_shape=jax.ShapeDtypeStruct((B, NC, HF, WF), jnp.float32),
    )(fine_features, aux4, w1p, W2, W3, b1r, b2r, b3r)

    return out
